# baseline (speedup 1.0000x reference)
import jax
import jax.numpy as jnp
from jax import lax
from jax.experimental import pallas as pl
from jax.experimental.pallas import tpu as pltpu
from jax.experimental.pallas import tpu_sc as plsc

VOCAB = 100000
EMBED = 2
BATCH = 1024

NUM_WORKERS = 32
BPW = BATCH // NUM_WORKERS
LANES = 16
ROW = 128
TAB_ROWS = (VOCAB * EMBED + ROW - 1) // ROW
VT = 2048


def _gather_body(x_hbm, tab_hbm, e_hbm, idx_v, eidx_v, ebuf_v, sem):
    wid = lax.axis_index("s") * 2 + lax.axis_index("c")
    base = wid * BPW
    pltpu.sync_copy(x_hbm.at[pl.ds(base, BPW)], idx_v)
    for g in range(BPW // LANES):
        idx16 = idx_v[pl.ds(g * LANES, LANES)]
        eidx_v[pl.ds(g * LANES, LANES)] = idx16 << 1
        eidx_v[pl.ds(BPW + g * LANES, LANES)] = (idx16 << 1) + 1
    pltpu.async_copy(tab_hbm.at[eidx_v], ebuf_v, sem).wait()
    pltpu.sync_copy(ebuf_v.at[pl.ds(0, BPW)], e_hbm.at[pl.ds(base, BPW)])
    pltpu.sync_copy(ebuf_v.at[pl.ds(BPW, BPW)],
                    e_hbm.at[pl.ds(BATCH + base, BPW)])


def _sc_gather(x, tab_flat):
    mesh = plsc.VectorSubcoreMesh(core_axis_name="c", subcore_axis_name="s")
    k = pl.kernel(
        _gather_body,
        out_type=jax.ShapeDtypeStruct((BATCH * EMBED,), jnp.float32),
        mesh=mesh,
        scratch_types=[
            pltpu.VMEM((BPW,), jnp.int32),
            pltpu.VMEM((BPW * EMBED,), jnp.int32),
            pltpu.VMEM((BPW * EMBED,), jnp.float32),
            pltpu.SemaphoreType.DMA,
        ],
    )
    return k(x, tab_flat)


VS = 3200
CT = 781
RB = 8
NBLK = BATCH // RB
ESPL = RB * LANES


def _scproj_body(et_hbm, w0_hbm, w1_hbm, b_hbm, out_hbm,
                 w0c, w1c, bc, rowbuf, eidx, espl, sems, esems):
    wid = lax.axis_index("s") * 2 + lax.axis_index("c")
    off = jnp.minimum(wid * (VS // 128), CT - (VS // 128)) * 128
    pltpu.sync_copy(w0_hbm.at[pl.ds(off, VS)], w0c)
    pltpu.sync_copy(w1_hbm.at[pl.ds(off, VS)], w1c)
    pltpu.sync_copy(b_hbm.at[pl.ds(off, VS)], bc)

    def _prefetch(kk, slot):
        for q in range(RB):
            base = kk * RB + q
            eidx[slot, 0, pl.ds(q * LANES, LANES)] = jnp.full(
                (LANES,), base, jnp.int32)
            eidx[slot, 1, pl.ds(q * LANES, LANES)] = jnp.full(
                (LANES,), base + BATCH, jnp.int32)
        pltpu.async_copy(et_hbm.at[eidx.at[slot, 0]], espl.at[slot, 0],
                         esems.at[slot]).start()
        pltpu.async_copy(et_hbm.at[eidx.at[slot, 1]], espl.at[slot, 1],
                         esems.at[slot]).start()

    _prefetch(0, 0)
    _prefetch(1, 1)

    def _block(k, _):
        slot = lax.rem(k, 2)

        @pl.when(k >= 2)
        def _wait_prev():
            pltpu.make_async_copy(
                rowbuf.at[slot],
                out_hbm.at[pl.ds(0, RB), pl.ds(0, VS)],
                sems.at[slot],
            ).wait()

        for _ in range(2):
            pltpu.make_async_copy(
                et_hbm.at[pl.ds(0, ESPL)], espl.at[slot, 0], esems.at[slot]
            ).wait()
        a0 = [espl[slot, 0, pl.ds(q * LANES, LANES)] for q in range(RB)]
        a1 = [espl[slot, 1, pl.ds(q * LANES, LANES)] for q in range(RB)]

        def _col(j, _):
            sl = pl.ds(j * LANES, LANES)
            w0j = w0c[sl]
            w1j = w1c[sl]
            bj = bc[sl]
            for q in range(RB):
                rowbuf[slot, q, sl] = a0[q] * w0j + a1[q] * w1j + bj
            return 0

        lax.fori_loop(0, VS // LANES, _col, 0)
        pltpu.make_async_copy(
            rowbuf.at[slot],
            out_hbm.at[pl.ds(k * RB, RB), pl.ds(off, VS)],
            sems.at[slot],
        ).start()

        @pl.when(k < NBLK - 2)
        def _next():
            _prefetch(k + 2, slot)

        return 0

    lax.fori_loop(0, NBLK, _block, 0)

    for s in range(2):
        pltpu.make_async_copy(
            rowbuf.at[s],
            out_hbm.at[pl.ds(0, RB), pl.ds(0, VS)],
            sems.at[s],
        ).wait()


def _sc_project(e_t, w0, w1, b):
    mesh = plsc.VectorSubcoreMesh(core_axis_name="c", subcore_axis_name="s")
    k = pl.kernel(
        _scproj_body,
        out_type=jax.ShapeDtypeStruct((BATCH, VOCAB), jnp.float32),
        mesh=mesh,
        scratch_types=[
            pltpu.VMEM((VS,), jnp.float32),
            pltpu.VMEM((VS,), jnp.float32),
            pltpu.VMEM((VS,), jnp.float32),
            pltpu.VMEM((2, RB, VS), jnp.float32),
            pltpu.VMEM((2, 2, ESPL), jnp.int32),
            pltpu.VMEM((2, 2, ESPL), jnp.float32),
            pltpu.SemaphoreType.DMA((2,)),
            pltpu.SemaphoreType.DMA((2,)),
        ],
    )
    return k(e_t, w0, w1, b)


EDGE_TILE = CT


def _edge_body(prev_ref, e_ref, wt_ref, b_ref, out_ref):
    del prev_ref
    e = e_ref[...]
    out_ref[...] = (
        e[:, 0:1] * wt_ref[0:1, :]
        + e[:, 1:2] * wt_ref[1:2, :]
        + b_ref[...]
    )


def _edge_fix(logits, e, wt, b2):
    return pl.pallas_call(
        _edge_body,
        grid=(1,),
        in_specs=[
            pl.BlockSpec(memory_space=pltpu.MemorySpace.HBM),
            pl.BlockSpec((BATCH, EMBED), lambda i: (0, 0)),
            pl.BlockSpec((EMBED, 128), lambda i: (0, EDGE_TILE)),
            pl.BlockSpec((1, 128), lambda i: (0, EDGE_TILE)),
        ],
        out_specs=pl.BlockSpec((BATCH, 128), lambda i: (0, EDGE_TILE)),
        out_shape=jax.ShapeDtypeStruct((BATCH, VOCAB), jnp.float32),
        input_output_aliases={0: 0},
    )(logits, e, wt, b2)


def kernel(x, table, W, b):
    x = x.astype(jnp.int32)
    e_t = _sc_gather(x, table.reshape(-1))
    e = e_t.reshape(EMBED, BATCH).T
    wt = W.T
    logits = _sc_project(e_t, wt[0], wt[1], b)
    logits = _edge_fix(logits, e, wt, b.reshape(1, VOCAB))
    return (logits, e)


# speedup vs baseline: 7.8655x; 7.8655x over previous
import jax
import jax.numpy as jnp
from jax import lax
from jax.experimental import pallas as pl
from jax.experimental.pallas import tpu as pltpu
from jax.experimental.pallas import tpu_sc as plsc

VOCAB = 100000
EMBED = 2
BATCH = 1024

NUM_WORKERS = 32
BPW = BATCH // NUM_WORKERS
LANES = 16
VT = 2048


def _gather_body(x_hbm, tab_hbm, e_hbm, idx_v, eidx_v, ebuf_v, sem):
    wid = lax.axis_index("s") * 2 + lax.axis_index("c")
    base = wid * BPW
    pltpu.sync_copy(x_hbm.at[pl.ds(base, BPW)], idx_v)
    for g in range(BPW // LANES):
        idx16 = idx_v[pl.ds(g * LANES, LANES)]
        eidx_v[pl.ds(g * LANES, LANES)] = idx16 << 1
        eidx_v[pl.ds(BPW + g * LANES, LANES)] = (idx16 << 1) + 1
    pltpu.async_copy(tab_hbm.at[eidx_v], ebuf_v, sem).wait()
    pltpu.sync_copy(ebuf_v.at[pl.ds(0, BPW)], e_hbm.at[pl.ds(base, BPW)])
    pltpu.sync_copy(ebuf_v.at[pl.ds(BPW, BPW)],
                    e_hbm.at[pl.ds(BATCH + base, BPW)])


def _sc_gather(x, tab_flat):
    mesh = plsc.VectorSubcoreMesh(core_axis_name="c", subcore_axis_name="s")
    k = pl.kernel(
        _gather_body,
        out_type=jax.ShapeDtypeStruct((BATCH * EMBED,), jnp.float32),
        mesh=mesh,
        scratch_types=[
            pltpu.VMEM((BPW,), jnp.int32),
            pltpu.VMEM((BPW * EMBED,), jnp.int32),
            pltpu.VMEM((BPW * EMBED,), jnp.float32),
            pltpu.SemaphoreType.DMA,
        ],
    )
    return k(x, tab_flat)


def _proj_body(e_ref, wt_ref, b_ref, out_ref):
    e = e_ref[...]
    out_ref[...] = (
        e[:, 0:1] * wt_ref[0:1, :]
        + e[:, 1:2] * wt_ref[1:2, :]
        + b_ref[...]
    )


def _project(e, wt, b2):
    return pl.pallas_call(
        _proj_body,
        grid=(pl.cdiv(VOCAB, VT),),
        in_specs=[
            pl.BlockSpec((BATCH, EMBED), lambda j: (0, 0)),
            pl.BlockSpec((EMBED, VT), lambda j: (0, j)),
            pl.BlockSpec((1, VT), lambda j: (0, j)),
        ],
        out_specs=pl.BlockSpec((BATCH, VT), lambda j: (0, j)),
        out_shape=jax.ShapeDtypeStruct((BATCH, VOCAB), jnp.float32),
    )(e, wt, b2)


def kernel(x, table, W, b):
    x = x.astype(jnp.int32)
    e_t = _sc_gather(x, table.reshape(-1))
    e = e_t.reshape(EMBED, BATCH).T
    logits = _project(e, W.T, b.reshape(1, VOCAB))
    return (logits, e)
